# baseline (device time: 43061 ns/iter reference)
import jax
import jax.numpy as jnp
from jax import lax
from jax.experimental import pallas as pl
from jax.experimental.pallas import tpu as pltpu

N_DEV = 4
N_LAYERS = 3


def kernel(x, Win0, Wout0, Win1, Wout1, Win2, Wout2):
    m_per, d = x.shape

    def body(
        x_ref,
        win0_ref,
        wout0_ref,
        win1_ref,
        wout1_ref,
        win2_ref,
        wout2_ref,
        out_ref,
        xg,
        pg,
        pmat,
        stage_in,
        stage_out,
        wbf_in,
        wbf_out,
        ag_send_sems,
        ag_recv_sems,
        rs_send_sems,
        rs_recv_sems,
        w_sems,
    ):
        my = lax.axis_index("i")

        def peer(k):
            return lax.rem(my + k, N_DEV)

        wins = [win0_ref, win1_ref, win2_ref]
        wouts = [wout0_ref, wout1_ref, wout2_ref]

        def start_w_dma(l):
            a = pltpu.make_async_copy(wins[l], stage_in, w_sems.at[0])
            b = pltpu.make_async_copy(wouts[l], stage_out, w_sems.at[1])
            a.start()
            b.start()
            return (a, b)

        w_dma = start_w_dma(0)

        barrier_sem = pltpu.get_barrier_semaphore()
        for k in (1, 2, 3):
            pl.semaphore_signal(
                barrier_sem, inc=1,
                device_id=(peer(k),), device_id_type=pl.DeviceIdType.MESH,
            )
        pl.semaphore_wait(barrier_sem, N_DEV - 1)

        def ag_recv_desc(j):
            return pltpu.make_async_remote_copy(
                src_ref=xg.at[j],
                dst_ref=xg.at[j],
                send_sem=ag_send_sems.at[0],
                recv_sem=ag_recv_sems.at[j],
                device_id=(my,),
                device_id_type=pl.DeviceIdType.MESH,
            )

        def rs_recv_desc(j):
            return pltpu.make_async_remote_copy(
                src_ref=pg.at[j],
                dst_ref=pg.at[j],
                send_sem=rs_send_sems.at[0],
                recv_sem=rs_recv_sems.at[j],
                device_id=(my,),
                device_id_type=pl.DeviceIdType.MESH,
            )

        def cast_weights():
            step = 512
            for r in range(0, d, step):
                wbf_in[r:r + step, :] = (
                    stage_in[r:r + step, :].astype(jnp.bfloat16)
                )
            for r in range(0, 2 * d, step):
                wbf_out[r:r + step, :] = (
                    stage_out[r:r + step, :].astype(jnp.bfloat16)
                )

        ag_prev = []
        rs_prev = {}

        x_loc = x_ref[:, :].astype(jnp.bfloat16)
        for l in range(N_LAYERS):
            for r in ag_prev:
                r.wait_send()
            xg[0, :, :] = x_loc
            ag_prev = []
            for k in (2, 1, 3):
                rdma = pltpu.make_async_remote_copy(
                    src_ref=xg.at[0],
                    dst_ref=xg.at[(N_DEV - k) % N_DEV],
                    send_sem=ag_send_sems.at[k],
                    recv_sem=ag_recv_sems.at[(N_DEV - k) % N_DEV],
                    device_id=(peer(k),),
                    device_id_type=pl.DeviceIdType.MESH,
                )
                rdma.start()
                ag_prev.append(rdma)

            w_dma[0].wait()
            w_dma[1].wait()
            cast_weights()
            if l + 1 < N_LAYERS:
                w_dma = start_w_dma(l + 1)

            def f(x_chunk):
                h = jnp.maximum(
                    jnp.dot(x_chunk, wbf_in[:, :],
                            preferred_element_type=jnp.float32),
                    0.0,
                ).astype(jnp.bfloat16)
                return jnp.dot(h, wbf_out[:, :],
                               preferred_element_type=jnp.float32)

            for j in (1, 3, 2):
                ag_recv_desc(j).wait_recv()
            x4 = xg[:, :, :].reshape(N_DEV * m_per, d)
            p4 = f(x4)
            p_own = p4[0:m_per, :]
            for r in rs_prev.values():
                r.wait_send()
            pmat[:, :] = p4.astype(jnp.bfloat16)

            rs_cur = {}
            for j in (2, 1, 3):
                rdma = pltpu.make_async_remote_copy(
                    src_ref=pmat.at[pl.ds(j * m_per, m_per)],
                    dst_ref=pg.at[(N_DEV - j) % N_DEV],
                    send_sem=rs_send_sems.at[j],
                    recv_sem=rs_recv_sems.at[(N_DEV - j) % N_DEV],
                    device_id=(peer(j),),
                    device_id_type=pl.DeviceIdType.MESH,
                )
                rdma.start()
                rs_cur[j] = rdma
            rs_prev = rs_cur

            acc = p_own
            for j in (3, 1, 2):
                rs_recv_desc(j).wait_recv()
                acc = acc + pg[j, :, :].astype(jnp.float32)
            x_loc = acc.astype(jnp.bfloat16)

        out_ref[:, :] = acc

        for r in ag_prev:
            r.wait_send()
        for r in rs_prev.values():
            r.wait_send()

    weight_spec = pl.BlockSpec(memory_space=pl.ANY)
    return pl.pallas_call(
        body,
        out_shape=jax.ShapeDtypeStruct((m_per, d), jnp.float32),
        in_specs=[pl.BlockSpec(memory_space=pltpu.VMEM)] + [weight_spec] * 6,
        out_specs=pl.BlockSpec(memory_space=pltpu.VMEM),
        scratch_shapes=[
            pltpu.VMEM((N_DEV, m_per, d), jnp.bfloat16),
            pltpu.VMEM((N_DEV, m_per, d), jnp.bfloat16),
            pltpu.VMEM((N_DEV * m_per, d), jnp.bfloat16),
            pltpu.VMEM((d, 2 * d), jnp.float32),
            pltpu.VMEM((2 * d, d), jnp.float32),
            pltpu.VMEM((d, 2 * d), jnp.bfloat16),
            pltpu.VMEM((2 * d, d), jnp.bfloat16),
            pltpu.SemaphoreType.DMA((N_DEV,)),
            pltpu.SemaphoreType.DMA((N_DEV,)),
            pltpu.SemaphoreType.DMA((N_DEV,)),
            pltpu.SemaphoreType.DMA((N_DEV,)),
            pltpu.SemaphoreType.DMA((2,)),
        ],
        compiler_params=pltpu.CompilerParams(
            collective_id=0,
            vmem_limit_bytes=100 * 1024 * 1024,
        ),
    )(x, Win0, Wout0, Win1, Wout1, Win2, Wout2)
